# baseline (speedup 1.0000x reference)
import functools

import jax
import jax.numpy as jnp
from jax import lax
from jax.experimental import pallas as pl
from jax.experimental.pallas import tpu as pltpu
from jax.experimental.pallas import tpu_sc as plsc


def _dist_argmax_body(x_ref, et_ref, dist_ref, ind_ref):
    d = jnp.dot(x_ref[...], et_ref[...], preferred_element_type=jnp.float32)
    dist_ref[...] = d
    idx = jnp.argmax(d, axis=1).astype(jnp.int32)
    ind_ref[0, 0, :] = idx


@functools.partial(jax.jit, static_argnames=("row_blk",))
def _dist_argmax(flat_x, embed_t, row_blk=128):
    n, d = flat_x.shape
    c = embed_t.shape[1]
    nblk = n // row_blk
    dist, ind3 = pl.pallas_call(
        _dist_argmax_body,
        grid=(nblk,),
        in_specs=[
            pl.BlockSpec((row_blk, d), lambda i: (i, 0)),
            pl.BlockSpec((d, c), lambda i: (0, 0)),
        ],
        out_specs=[
            pl.BlockSpec((row_blk, c), lambda i: (i, 0)),
            pl.BlockSpec((1, 1, row_blk), lambda i: (i, 0, 0)),
        ],
        out_shape=[
            jax.ShapeDtypeStruct((n, c), jnp.float32),
            jax.ShapeDtypeStruct((nblk, 1, row_blk), jnp.int32),
        ],
    )(flat_x, embed_t)
    return dist, ind3.reshape(n)


def _make_sc_gather(v, d, b):
    nc, ns = 2, 16
    nw = nc * ns
    assert b % (8 * nw) == 0 and d % 16 == 0
    b_per_w = b // nw
    mesh = plsc.VectorSubcoreMesh(core_axis_name="c", subcore_axis_name="s")

    @functools.partial(
        pl.kernel,
        mesh=mesh,
        out_type=jax.ShapeDtypeStruct((b, d), jnp.float32),
        scratch_types=[
            pltpu.VMEM((b_per_w,), jnp.int32),
            pltpu.VMEM((b_per_w, d), jnp.float32),
            pltpu.SemaphoreType.DMA,
        ],
        compiler_params=pltpu.CompilerParams(use_tc_tiling_on_sc=False),
    )
    def gather(table_hbm, idx_hbm, out_hbm, idx_v, rows_v, sem):
        wid = lax.axis_index("s") * nc + lax.axis_index("c")
        base = wid * b_per_w
        pltpu.sync_copy(idx_hbm.at[pl.ds(base, b_per_w)], idx_v)
        pltpu.async_copy(table_hbm.at[idx_v], rows_v, sem).wait()
        pltpu.sync_copy(rows_v, out_hbm.at[pl.ds(base, b_per_w)])

    return gather


def kernel(x, embed):
    b, n, d = x.shape
    c = embed.shape[1]
    flat = x.astype(jnp.float32).reshape(b * n, d)
    table = embed[0].astype(jnp.float32)

    dist, ind = _dist_argmax(flat, table.T)
    quantize = _make_sc_gather(c, d, b * n)(table, ind)

    return (
        quantize.reshape(b, n, d),
        ind.reshape(b, n),
        dist.reshape(1, b, n, c),
    )


# speedup vs baseline: 1.0661x; 1.0661x over previous
import functools

import jax
import jax.numpy as jnp
from jax import lax
from jax.experimental import pallas as pl
from jax.experimental.pallas import tpu as pltpu
from jax.experimental.pallas import tpu_sc as plsc


def _dist_argmax_body(x_ref, emb_ref, dist_ref, ind_ref):
    d = jax.lax.dot_general(
        x_ref[...], emb_ref[...],
        dimension_numbers=(((1,), (1,)), ((), ())),
        preferred_element_type=jnp.float32,
    )
    dist_ref[...] = d
    idx = jnp.argmax(d, axis=1).astype(jnp.int32)
    ind_ref[0, 0, :] = idx


@functools.partial(jax.jit, static_argnames=("row_blk",))
def _dist_argmax(flat_x, embed, row_blk=256):
    n, d = flat_x.shape
    c = embed.shape[0]
    nblk = n // row_blk
    dist, ind3 = pl.pallas_call(
        _dist_argmax_body,
        grid=(nblk,),
        in_specs=[
            pl.BlockSpec((row_blk, d), lambda i: (i, 0)),
            pl.BlockSpec((c, d), lambda i: (0, 0)),
        ],
        out_specs=[
            pl.BlockSpec((row_blk, c), lambda i: (i, 0)),
            pl.BlockSpec((1, 1, row_blk), lambda i: (i, 0, 0)),
        ],
        out_shape=[
            jax.ShapeDtypeStruct((n, c), jnp.float32),
            jax.ShapeDtypeStruct((nblk, 1, row_blk), jnp.int32),
        ],
    )(flat_x, embed)
    return dist, ind3.reshape(n)


def _make_sc_gather(v, d, b):
    nc, ns = 2, 16
    nw = nc * ns
    assert b % (8 * nw) == 0 and d % 16 == 0
    b_per_w = b // nw
    mesh = plsc.VectorSubcoreMesh(core_axis_name="c", subcore_axis_name="s")

    @functools.partial(
        pl.kernel,
        mesh=mesh,
        out_type=jax.ShapeDtypeStruct((b, d), jnp.float32),
        scratch_types=[
            pltpu.VMEM((b_per_w,), jnp.int32),
            pltpu.VMEM((b_per_w, d), jnp.float32),
            pltpu.SemaphoreType.DMA,
        ],
        compiler_params=pltpu.CompilerParams(use_tc_tiling_on_sc=False),
    )
    def gather(table_hbm, idx_hbm, out_hbm, idx_v, rows_v, sem):
        wid = lax.axis_index("s") * nc + lax.axis_index("c")
        base = wid * b_per_w
        pltpu.sync_copy(idx_hbm.at[pl.ds(base, b_per_w)], idx_v)
        pltpu.async_copy(table_hbm.at[idx_v], rows_v, sem).wait()
        pltpu.sync_copy(rows_v, out_hbm.at[pl.ds(base, b_per_w)])

    return gather


def kernel(x, embed):
    b, n, d = x.shape
    c = embed.shape[1]
    flat = x.astype(jnp.float32).reshape(b * n, d)
    table = embed[0].astype(jnp.float32)

    dist, ind = _dist_argmax(flat, table)
    quantize = _make_sc_gather(c, d, b * n)(table, ind)

    return (
        quantize.reshape(b, n, d),
        ind.reshape(b, n),
        dist.reshape(1, b, n, c),
    )


# speedup vs baseline: 1.0822x; 1.0151x over previous
import functools

import jax
import jax.numpy as jnp
from jax import lax
from jax.experimental import pallas as pl
from jax.experimental.pallas import tpu as pltpu
from jax.experimental.pallas import tpu_sc as plsc


def _dist_argmax_body(x_ref, et_ref, dist_ref, ind_ref):
    r = x_ref.shape[1]
    xb = x_ref[...].reshape(r, x_ref.shape[2])
    d = jnp.dot(xb, et_ref[...], preferred_element_type=jnp.float32)
    dist_ref[...] = d.reshape(dist_ref.shape)
    idx = jnp.argmax(d, axis=1).astype(jnp.int32)
    ind_ref[...] = idx


@functools.partial(jax.jit, static_argnames=("row_blk",))
def _dist_argmax(x, embed_t, row_blk=256):
    b, n, d = x.shape
    c = embed_t.shape[1]
    nblk = (b * n) // row_blk
    per_b = n // row_blk
    dist, ind = pl.pallas_call(
        _dist_argmax_body,
        grid=(nblk,),
        in_specs=[
            pl.BlockSpec((1, row_blk, d), lambda i: (i // per_b, i % per_b, 0)),
            pl.BlockSpec((d, c), lambda i: (0, 0)),
        ],
        out_specs=[
            pl.BlockSpec(
                (1, 1, row_blk, c), lambda i: (0, i // per_b, i % per_b, 0)
            ),
            pl.BlockSpec((row_blk,), lambda i: (i,)),
        ],
        out_shape=[
            jax.ShapeDtypeStruct((1, b, n, c), jnp.float32),
            jax.ShapeDtypeStruct((b * n,), jnp.int32),
        ],
    )(x, embed_t)
    return dist, ind


def _make_sc_gather(v, d, bb, nn):
    nc, ns = 2, 16
    nw = nc * ns
    b = bb * nn
    assert b % (8 * nw) == 0 and d % 16 == 0
    b_per_w = b // nw
    w_per_b = nn // b_per_w
    mesh = plsc.VectorSubcoreMesh(core_axis_name="c", subcore_axis_name="s")

    @functools.partial(
        pl.kernel,
        mesh=mesh,
        out_type=jax.ShapeDtypeStruct((bb, nn, d), jnp.float32),
        scratch_types=[
            pltpu.VMEM((b_per_w,), jnp.int32),
            pltpu.VMEM((b_per_w, d), jnp.float32),
            pltpu.SemaphoreType.DMA,
        ],
        compiler_params=pltpu.CompilerParams(use_tc_tiling_on_sc=False),
    )
    def gather(table_hbm, idx_hbm, out_hbm, idx_v, rows_v, sem):
        wid = lax.axis_index("s") * nc + lax.axis_index("c")
        base = wid * b_per_w
        pltpu.sync_copy(idx_hbm.at[pl.ds(base, b_per_w)], idx_v)
        pltpu.async_copy(table_hbm.at[idx_v], rows_v, sem).wait()
        pltpu.sync_copy(
            rows_v, out_hbm.at[wid // w_per_b, pl.ds((wid % w_per_b) * b_per_w, b_per_w)]
        )

    return gather


def kernel(x, embed):
    b, n, d = x.shape
    c = embed.shape[1]
    table = embed[0].astype(jnp.float32)

    dist, ind = _dist_argmax(x.astype(jnp.float32), table.T)
    quantize = _make_sc_gather(c, d, b, n)(table, ind)

    return (quantize, ind.reshape(b, n), dist)
